# Initial kernel scaffold; baseline (speedup 1.0000x reference)
import jax
import jax.numpy as jnp
from jax.experimental import pallas as pl


def kernel(obs, rm0_node_features, rm0_edge_index, rm0_edge_features, obs_W1, obs_b1, obs_W2, obs_b2, obs_W3, obs_b3, gcn_W1, gcn_b1, gcn_W2, gcn_b2, mlp_W1, mlp_b1, mlp_W2, mlp_b2, mlp_W3, mlp_b3):
    raise NotImplementedError("write your pallas kernel here")


# speedup vs baseline: 27.2261x; 27.2261x over previous
import functools

import jax
import jax.numpy as jnp
from jax import lax
from jax.experimental import pallas as pl
from jax.experimental.pallas import tpu as pltpu
from jax.experimental.pallas import tpu_sc as plsc

N_NODES = 10000
N_EDGES = 320000
D_FEAT = 128
D_HID = 16
NC = 2
NS = 16
N_TILES = NC * NS
E_PER_TILE = N_EDGES // N_TILES
CHUNK = 2000
N_CHUNKS = E_PER_TILE // CHUNK
ZROWS = N_NODES // 10


def _hlin_body(x_ref, w_ref, o_ref):
    o_ref[...] = jnp.dot(x_ref[...], w_ref[...],
                         preferred_element_type=jnp.float32)


def _hlin(x, w1):
    return pl.pallas_call(
        _hlin_body,
        out_shape=jax.ShapeDtypeStruct((N_NODES, D_HID), jnp.float32),
    )(x, w1)


def _edge_body(hlin_hbm, src_hbm, dst_hbm, ew_hbm, z2_hbm, z1_hbm,
               agg_out, wout_out,
               src_v, dst_v, ew_v, rows_v, agg_sp, wout_sp, sem):
    c = lax.axis_index("c")
    s = lax.axis_index("s")
    wid = c * NS + s

    @pl.when(s < 10)
    def _():
        pltpu.sync_copy(z2_hbm.at[pl.ds(s * ZROWS, ZROWS)],
                        rows_v.at[pl.ds(0, ZROWS)])
        pltpu.sync_copy(rows_v.at[pl.ds(0, ZROWS)],
                        agg_sp.at[pl.ds(s * ZROWS, ZROWS)])
        pltpu.sync_copy(z1_hbm.at[pl.ds(s * ZROWS, ZROWS)],
                        ew_v.at[pl.ds(0, ZROWS)])
        pltpu.sync_copy(ew_v.at[pl.ds(0, ZROWS)],
                        wout_sp.at[pl.ds(s * ZROWS, ZROWS)])

    plsc.subcore_barrier()

    for k in range(N_CHUNKS):
        base = wid * E_PER_TILE + k * CHUNK
        pltpu.sync_copy(src_hbm.at[pl.ds(base, CHUNK)], src_v)
        pltpu.sync_copy(dst_hbm.at[pl.ds(base, CHUNK)], dst_v)
        pltpu.sync_copy(ew_hbm.at[pl.ds(base, CHUNK)], ew_v)
        pltpu.async_copy(hlin_hbm.at[src_v], rows_v, sem).wait()

        def _scale(g, _):
            base_e = g * 16
            ewv = ew_v[pl.ds(base_e, 16)]
            for j in range(16):
                rows_v[base_e + j, :] = rows_v[base_e + j, :] * ewv[j]
            return 0
        lax.fori_loop(0, CHUNK // 16, _scale, 0)

        pltpu.sync_copy(rows_v, agg_sp.at[dst_v], add=True)
        pltpu.sync_copy(ew_v, wout_sp.at[src_v], add=True)

    plsc.subcore_barrier()

    @pl.when(s < 10)
    def _():
        pltpu.sync_copy(agg_sp.at[pl.ds(s * ZROWS, ZROWS)],
                        rows_v.at[pl.ds(0, ZROWS)])
        pltpu.sync_copy(rows_v.at[pl.ds(0, ZROWS)],
                        agg_out.at[c, pl.ds(s * ZROWS, ZROWS)])
        pltpu.sync_copy(wout_sp.at[pl.ds(s * ZROWS, ZROWS)],
                        ew_v.at[pl.ds(0, ZROWS)])
        pltpu.sync_copy(ew_v.at[pl.ds(0, ZROWS)],
                        wout_out.at[pl.ds(c * N_NODES + s * ZROWS, ZROWS)])


def _edge_pass(hlin, src, dst, ew, z2, z1):
    mesh = plsc.VectorSubcoreMesh(core_axis_name="c", subcore_axis_name="s")
    f = pl.kernel(
        _edge_body,
        out_type=(jax.ShapeDtypeStruct((NC, N_NODES, D_HID), jnp.float32),
                  jax.ShapeDtypeStruct((NC * N_NODES,), jnp.float32)),
        mesh=mesh,
        scratch_types=[
            pltpu.VMEM((CHUNK,), jnp.int32),
            pltpu.VMEM((CHUNK,), jnp.int32),
            pltpu.VMEM((CHUNK,), jnp.float32),
            pltpu.VMEM((CHUNK, D_HID), jnp.float32),
            pltpu.VMEM_SHARED((N_NODES, D_HID), jnp.float32),
            pltpu.VMEM_SHARED((N_NODES,), jnp.float32),
            pltpu.SemaphoreType.DMA,
        ],
        compiler_params=pltpu.CompilerParams(use_tc_tiling_on_sc=False),
    )
    return f(hlin, src, dst, ew, z2, z1)


def _finish_body(agg_ref, wout_ref, obs_ref,
                 ow1, ob1, ow2, ob2, ow3, ob3,
                 gb1, gw2, gb2,
                 mw1, mb1, mw2, mb2, mw3, mb3,
                 y_ref):
    relu = lambda v: jnp.maximum(v, 0.0)
    agg = agg_ref[0] + agg_ref[1] + gb1[...]
    h1 = relu(agg)
    wout = wout_ref[0:1, :] + wout_ref[1:2, :]
    s16 = jnp.dot(wout, h1, preferred_element_type=jnp.float32)
    g = jnp.dot(s16 / float(N_NODES), gw2[...],
                preferred_element_type=jnp.float32) + gb2[...]

    o = relu(jnp.dot(obs_ref[...], ow1[...],
                     preferred_element_type=jnp.float32) + ob1[...])
    o = relu(jnp.dot(o, ow2[...], preferred_element_type=jnp.float32) + ob2[...])
    o = jnp.dot(o, ow3[...], preferred_element_type=jnp.float32) + ob3[...]

    f = jnp.concatenate([o, g], axis=1)
    y = relu(jnp.dot(f, mw1[...], preferred_element_type=jnp.float32) + mb1[...])
    y = relu(jnp.dot(y, mw2[...], preferred_element_type=jnp.float32) + mb2[...])
    y_ref[...] = jnp.dot(y, mw3[...], preferred_element_type=jnp.float32) + mb3[...]


def _finish(agg_parts, wout_parts, obs2d, ow1, ob1, ow2, ob2, ow3, ob3,
            gb1, gw2, gb2, mw1, mb1, mw2, mb2, mw3, mb3):
    return pl.pallas_call(
        _finish_body,
        out_shape=jax.ShapeDtypeStruct((1, 8), jnp.float32),
    )(agg_parts, wout_parts, obs2d, ow1, ob1, ow2, ob2, ow3, ob3,
      gb1, gw2, gb2, mw1, mb1, mw2, mb2, mw3, mb3)


def kernel(obs, rm0_node_features, rm0_edge_index, rm0_edge_features,
           obs_W1, obs_b1, obs_W2, obs_b2, obs_W3, obs_b3,
           gcn_W1, gcn_b1, gcn_W2, gcn_b2,
           mlp_W1, mlp_b1, mlp_W2, mlp_b2, mlp_W3, mlp_b3):
    src = rm0_edge_index[0].astype(jnp.int32)
    dst = rm0_edge_index[1].astype(jnp.int32)
    ew = rm0_edge_features[:, 0]
    z2 = jnp.zeros((N_NODES, D_HID), jnp.float32)
    z1 = jnp.zeros((N_NODES,), jnp.float32)

    hlin = _hlin(rm0_node_features, gcn_W1)
    agg_parts, wout_flat = _edge_pass(hlin, src, dst, ew, z2, z1)
    wout_parts = wout_flat.reshape(NC, N_NODES)
    y = _finish(agg_parts, wout_parts, obs.reshape(1, -1),
                obs_W1, obs_b1.reshape(1, -1), obs_W2, obs_b2.reshape(1, -1),
                obs_W3, obs_b3.reshape(1, -1),
                gcn_b1.reshape(1, -1), gcn_W2, gcn_b2.reshape(1, -1),
                mlp_W1, mlp_b1.reshape(1, -1), mlp_W2, mlp_b2.reshape(1, -1),
                mlp_W3, mlp_b3.reshape(1, -1))
    return y.reshape(8)


# speedup vs baseline: 28.7002x; 1.0541x over previous
import functools

import jax
import jax.numpy as jnp
from jax import lax
from jax.experimental import pallas as pl
from jax.experimental.pallas import tpu as pltpu
from jax.experimental.pallas import tpu_sc as plsc

N_NODES = 10000
N_EDGES = 320000
D_FEAT = 128
D_HID = 16
NC = 2
NS = 16
N_TILES = NC * NS
E_PER_TILE = N_EDGES // N_TILES
CHUNK = 2000
N_CHUNKS = E_PER_TILE // CHUNK
ZROWS = N_NODES // 10


def _hlin_body(x_ref, w_ref, o_ref):
    o_ref[...] = jnp.dot(x_ref[...], w_ref[...],
                         preferred_element_type=jnp.float32, precision=lax.Precision.HIGHEST)


def _hlin(x, w1):
    return pl.pallas_call(
        _hlin_body,
        out_shape=jax.ShapeDtypeStruct((N_NODES, D_HID), jnp.float32),
    )(x, w1)


def _edge_body(hlin_hbm, ei_hbm, ef_hbm, z2_hbm, z1_hbm,
               agg_out, wout_out,
               src_v, dst_v, ew_v, rows_v, agg_sp, wout_sp, sem):
    c = lax.axis_index("c")
    s = lax.axis_index("s")
    wid = c * NS + s

    @pl.when(s < 10)
    def _():
        pltpu.sync_copy(z2_hbm.at[pl.ds(s * ZROWS, ZROWS)],
                        rows_v.at[pl.ds(0, ZROWS)])
        pltpu.sync_copy(rows_v.at[pl.ds(0, ZROWS)],
                        agg_sp.at[pl.ds(s * ZROWS, ZROWS)])
        pltpu.sync_copy(z1_hbm.at[pl.ds(s * ZROWS, ZROWS)],
                        ew_v.at[pl.ds(0, ZROWS)])
        pltpu.sync_copy(ew_v.at[pl.ds(0, ZROWS)],
                        wout_sp.at[pl.ds(s * ZROWS, ZROWS)])

    plsc.subcore_barrier()

    for k in range(N_CHUNKS):
        base = wid * E_PER_TILE + k * CHUNK
        pltpu.sync_copy(ei_hbm.at[0, pl.ds(base, CHUNK)], src_v)
        pltpu.sync_copy(ei_hbm.at[1, pl.ds(base, CHUNK)], dst_v)
        pltpu.sync_copy(ef_hbm.at[pl.ds(base, CHUNK)], ew_v)
        pltpu.async_copy(hlin_hbm.at[src_v], rows_v, sem).wait()

        def _scale(g, _):
            base_e = g * 16
            ewv = ew_v[pl.ds(base_e, 16)]
            for j in range(16):
                rows_v[base_e + j, :] = rows_v[base_e + j, :] * ewv[j]
            return 0
        lax.fori_loop(0, CHUNK // 16, _scale, 0)

        pltpu.sync_copy(rows_v, agg_sp.at[dst_v], add=True)
        pltpu.sync_copy(ew_v, wout_sp.at[src_v], add=True)

    plsc.subcore_barrier()

    @pl.when(s < 10)
    def _():
        pltpu.sync_copy(agg_sp.at[pl.ds(s * ZROWS, ZROWS)],
                        rows_v.at[pl.ds(0, ZROWS)])
        pltpu.sync_copy(rows_v.at[pl.ds(0, ZROWS)],
                        agg_out.at[c, pl.ds(s * ZROWS, ZROWS)])
        pltpu.sync_copy(wout_sp.at[pl.ds(s * ZROWS, ZROWS)],
                        ew_v.at[pl.ds(0, ZROWS)])
        pltpu.sync_copy(ew_v.at[pl.ds(0, ZROWS)],
                        wout_out.at[c, pl.ds(s * ZROWS, ZROWS)])


def _edge_pass(hlin, ei, ef, z2, z1):
    mesh = plsc.VectorSubcoreMesh(core_axis_name="c", subcore_axis_name="s")
    f = pl.kernel(
        _edge_body,
        out_type=(jax.ShapeDtypeStruct((NC, N_NODES, D_HID), jnp.float32),
                  jax.ShapeDtypeStruct((NC, N_NODES), jnp.float32)),
        mesh=mesh,
        scratch_types=[
            pltpu.VMEM((CHUNK,), jnp.int32),
            pltpu.VMEM((CHUNK,), jnp.int32),
            pltpu.VMEM((CHUNK,), jnp.float32),
            pltpu.VMEM((CHUNK, D_HID), jnp.float32),
            pltpu.VMEM_SHARED((N_NODES, D_HID), jnp.float32),
            pltpu.VMEM_SHARED((N_NODES,), jnp.float32),
            pltpu.SemaphoreType.DMA,
        ],
        compiler_params=pltpu.CompilerParams(use_tc_tiling_on_sc=False),
    )
    return f(hlin, ei, ef, z2, z1)


def _finish_body(agg_ref, wout_ref, obs_ref,
                 ow1, ob1, ow2, ob2, ow3, ob3,
                 gb1, gw2, gb2,
                 mw1, mb1, mw2, mb2, mw3, mb3,
                 y_ref):
    relu = lambda v: jnp.maximum(v, 0.0)
    agg = agg_ref[0] + agg_ref[1] + gb1[...]
    h1 = relu(agg)
    wout = wout_ref[0:1, :] + wout_ref[1:2, :]
    s16 = jnp.dot(wout, h1, preferred_element_type=jnp.float32, precision=lax.Precision.HIGHEST)
    g = jnp.dot(s16 / float(N_NODES), gw2[...],
                preferred_element_type=jnp.float32, precision=lax.Precision.HIGHEST) + gb2[...]

    o = relu(jnp.dot(obs_ref[...], ow1[...],
                     preferred_element_type=jnp.float32, precision=lax.Precision.HIGHEST) + ob1[...])
    o = relu(jnp.dot(o, ow2[...], preferred_element_type=jnp.float32, precision=lax.Precision.HIGHEST) + ob2[...])
    o = jnp.dot(o, ow3[...], preferred_element_type=jnp.float32, precision=lax.Precision.HIGHEST) + ob3[...]

    f = jnp.concatenate([o, g], axis=1)
    y = relu(jnp.dot(f, mw1[...], preferred_element_type=jnp.float32, precision=lax.Precision.HIGHEST) + mb1[...])
    y = relu(jnp.dot(y, mw2[...], preferred_element_type=jnp.float32, precision=lax.Precision.HIGHEST) + mb2[...])
    y_ref[...] = jnp.dot(y, mw3[...], preferred_element_type=jnp.float32, precision=lax.Precision.HIGHEST) + mb3[...]


def _finish(agg_parts, wout_parts, obs2d, ow1, ob1, ow2, ob2, ow3, ob3,
            gb1, gw2, gb2, mw1, mb1, mw2, mb2, mw3, mb3):
    return pl.pallas_call(
        _finish_body,
        out_shape=jax.ShapeDtypeStruct((1, 8), jnp.float32),
    )(agg_parts, wout_parts, obs2d, ow1, ob1, ow2, ob2, ow3, ob3,
      gb1, gw2, gb2, mw1, mb1, mw2, mb2, mw3, mb3)


def kernel(obs, rm0_node_features, rm0_edge_index, rm0_edge_features,
           obs_W1, obs_b1, obs_W2, obs_b2, obs_W3, obs_b3,
           gcn_W1, gcn_b1, gcn_W2, gcn_b2,
           mlp_W1, mlp_b1, mlp_W2, mlp_b2, mlp_W3, mlp_b3):
    z2 = jnp.zeros((N_NODES, D_HID), jnp.float32)
    z1 = jnp.zeros((N_NODES,), jnp.float32)

    hlin = _hlin(rm0_node_features, gcn_W1)
    agg_parts, wout_parts = _edge_pass(hlin, rm0_edge_index,
                                       rm0_edge_features[:, 0], z2, z1)
    y = _finish(agg_parts, wout_parts, obs.reshape(1, -1),
                obs_W1, obs_b1.reshape(1, -1), obs_W2, obs_b2.reshape(1, -1),
                obs_W3, obs_b3.reshape(1, -1),
                gcn_b1.reshape(1, -1), gcn_W2, gcn_b2.reshape(1, -1),
                mlp_W1, mlp_b1.reshape(1, -1), mlp_W2, mlp_b2.reshape(1, -1),
                mlp_W3, mlp_b3.reshape(1, -1))
    return y.reshape(8)
